# baseline (speedup 1.0000x reference)
import jax
import jax.numpy as jnp
from jax import lax
from jax.experimental import pallas as pl
from jax.experimental.pallas import tpu as pltpu


def _conv_stats_body(H, W, FW, C, K, YL, taps):
    def body(x_ref, w_ref, y_ref, stats_ref, rhs_ref):
        for t, (kh, kw) in enumerate(taps):
            off = kh * FW + kw
            rhs_ref[pl.ds(t * C, C), :] = x_ref[0, :, pl.ds(off, YL)]
        y = jnp.dot(w_ref[...], rhs_ref[...],
                    preferred_element_type=jnp.float32)
        y_ref[0] = y
        lane = lax.broadcasted_iota(jnp.int32, (1, YL), 1)
        mask = ((lane % FW) < W).astype(jnp.float32)
        ym = y * mask
        s = jnp.sum(ym, axis=1, keepdims=True)
        sq = jnp.sum(ym * y, axis=1, keepdims=True)
        stats_ref[0] = jnp.concatenate([s, sq], axis=1)
    return body


def _bn_body(W, ):
    def body(y_ref, scale_ref, shift_ref, o_ref):
        o_ref[...] = (y_ref[..., :W] * scale_ref[...] + shift_ref[...])
    return body


def kernel(x_nchw, w_oihw, gamma, beta):
    eps = 1e-5
    N, C, H, W = x_nchw.shape
    Cout, _, KH, KW = w_oihw.shape
    FW = W + 8
    FH = H + 4
    XL = FH * FW
    YL = H * FW
    K = KH * KW * C
    taps = tuple((kh, kw) for kh in range(KH) for kw in range(KW))

    x = jnp.pad(x_nchw, ((0, 0), (0, 0), (1, FH - H - 1), (1, FW - W - 1)))
    x = x.reshape(N, C, XL)
    w = jnp.transpose(w_oihw, (0, 2, 3, 1)).reshape(Cout, K)

    y, stats = pl.pallas_call(
        _conv_stats_body(H, W, FW, C, K, YL, taps),
        out_shape=(jax.ShapeDtypeStruct((N, Cout, YL), jnp.float32),
                   jax.ShapeDtypeStruct((N, Cout, 2), jnp.float32)),
        grid=(N,),
        in_specs=[pl.BlockSpec((1, C, XL), lambda i: (i, 0, 0)),
                  pl.BlockSpec((Cout, K), lambda i: (0, 0))],
        out_specs=(pl.BlockSpec((1, Cout, YL), lambda i: (i, 0, 0)),
                   pl.BlockSpec((1, Cout, 2), lambda i: (i, 0, 0))),
        scratch_shapes=[pltpu.VMEM((K, YL), jnp.float32)],
        compiler_params=pltpu.CompilerParams(
            dimension_semantics=("parallel",),
            vmem_limit_bytes=64 * 1024 * 1024),
    )(x, w)

    m = N * H * W
    sums = jnp.sum(stats[:, :, 0], axis=0)
    sumsq = jnp.sum(stats[:, :, 1], axis=0)
    mean = sums / m
    var = jnp.maximum(sumsq / m - mean * mean, 0.0)
    scale = gamma.astype(jnp.float32) * lax.rsqrt(var + eps)
    shift = beta.astype(jnp.float32) - mean * scale

    B = next(b for b in (4, 2, 1) if N % b == 0)
    y4 = y.reshape(N, Cout, H, FW)
    out = pl.pallas_call(
        _bn_body(W),
        out_shape=jax.ShapeDtypeStruct((N, Cout, H, W), jnp.float32),
        grid=(N // B,),
        in_specs=[pl.BlockSpec((B, Cout, H, FW), lambda i: (i, 0, 0, 0)),
                  pl.BlockSpec((Cout, 1, 1), lambda i: (0, 0, 0)),
                  pl.BlockSpec((Cout, 1, 1), lambda i: (0, 0, 0))],
        out_specs=pl.BlockSpec((B, Cout, H, W), lambda i: (i, 0, 0, 0)),
        compiler_params=pltpu.CompilerParams(
            dimension_semantics=("parallel",),
            vmem_limit_bytes=64 * 1024 * 1024),
    )(y4, scale.reshape(Cout, 1, 1), shift.reshape(Cout, 1, 1))

    return out


# speedup vs baseline: 1.3697x; 1.3697x over previous
import jax
import jax.numpy as jnp
from jax import lax
from jax.experimental import pallas as pl
from jax.experimental.pallas import tpu as pltpu


def _conv_stats_body(H, W, FW, C, K, YL, taps):
    def body(x_ref, w_ref, y_ref, stats_ref, rhs_ref):
        for t, (kh, kw) in enumerate(taps):
            off = kh * FW + kw
            rhs_ref[pl.ds(t * C, C), :] = x_ref[0, :, pl.ds(off, YL)]
        y = jnp.dot(w_ref[...], rhs_ref[...],
                    preferred_element_type=jnp.float32)
        for r in range(H):
            y_ref[0, :, pl.ds(r * W, W)] = y[:, r * FW:r * FW + W]
        yc = y_ref[0]
        s = jnp.sum(yc, axis=1, keepdims=True)
        sq = jnp.sum(yc * yc, axis=1, keepdims=True)
        stats_ref[0] = jnp.concatenate([s, sq], axis=1)
    return body


def _bn_body(y_ref, scale_ref, shift_ref, o_ref):
    o_ref[...] = y_ref[...] * scale_ref[...] + shift_ref[...]


def kernel(x_nchw, w_oihw, gamma, beta):
    eps = 1e-5
    N, C, H, W = x_nchw.shape
    Cout, _, KH, KW = w_oihw.shape
    FW = W + 8
    FH = H + 4
    XL = FH * FW
    YL = H * FW
    PL = H * W
    K = KH * KW * C
    taps = tuple((kh, kw) for kh in range(KH) for kw in range(KW))

    x = jnp.pad(x_nchw, ((0, 0), (0, 0), (1, FH - H - 1), (1, FW - W - 1)))
    x = x.reshape(N, C, XL)
    w = jnp.transpose(w_oihw, (0, 2, 3, 1)).reshape(Cout, K)

    y, stats = pl.pallas_call(
        _conv_stats_body(H, W, FW, C, K, YL, taps),
        out_shape=(jax.ShapeDtypeStruct((N, Cout, PL), jnp.float32),
                   jax.ShapeDtypeStruct((N, Cout, 2), jnp.float32)),
        grid=(N,),
        in_specs=[pl.BlockSpec((1, C, XL), lambda i: (i, 0, 0)),
                  pl.BlockSpec((Cout, K), lambda i: (0, 0))],
        out_specs=(pl.BlockSpec((1, Cout, PL), lambda i: (i, 0, 0)),
                   pl.BlockSpec((1, Cout, 2), lambda i: (i, 0, 0))),
        scratch_shapes=[pltpu.VMEM((K, YL), jnp.float32)],
        compiler_params=pltpu.CompilerParams(
            dimension_semantics=("parallel",),
            vmem_limit_bytes=64 * 1024 * 1024),
    )(x, w)

    m = N * H * W
    sums = jnp.sum(stats[:, :, 0], axis=0)
    sumsq = jnp.sum(stats[:, :, 1], axis=0)
    mean = sums / m
    var = jnp.maximum(sumsq / m - mean * mean, 0.0)
    scale = gamma.astype(jnp.float32) * lax.rsqrt(var + eps)
    shift = beta.astype(jnp.float32) - mean * scale

    B = next(b for b in (4, 2, 1) if N % b == 0)
    out_flat = pl.pallas_call(
        _bn_body,
        out_shape=jax.ShapeDtypeStruct((N, Cout, PL), jnp.float32),
        grid=(N // B,),
        in_specs=[pl.BlockSpec((B, Cout, PL), lambda i: (i, 0, 0)),
                  pl.BlockSpec((Cout, 1), lambda i: (0, 0)),
                  pl.BlockSpec((Cout, 1), lambda i: (0, 0))],
        out_specs=pl.BlockSpec((B, Cout, PL), lambda i: (i, 0, 0)),
        compiler_params=pltpu.CompilerParams(
            dimension_semantics=("parallel",),
            vmem_limit_bytes=64 * 1024 * 1024),
    )(y, scale.reshape(Cout, 1), shift.reshape(Cout, 1))

    return out_flat.reshape(N, Cout, H, W)


# speedup vs baseline: 1.4278x; 1.0424x over previous
import jax
import jax.numpy as jnp
from jax import lax
from jax.experimental import pallas as pl
from jax.experimental.pallas import tpu as pltpu


def _conv_stats_body(H, W, FW, C, K, YL, taps):
    def body(x_ref, w_ref, y_ref, stats_ref, rhs_ref):
        for t, (kh, kw) in enumerate(taps):
            off = kh * FW + kw
            rhs_ref[pl.ds(t * C, C), :] = x_ref[0, :, pl.ds(off, YL)]
        y = jnp.dot(w_ref[...], rhs_ref[...],
                    preferred_element_type=jnp.float32)
        for r in range(H):
            y_ref[0, :, pl.ds(r * W, W)] = y[:, r * FW:r * FW + W]
        yc = y_ref[0]
        s = jnp.sum(yc, axis=1, keepdims=True)
        sq = jnp.sum(yc * yc, axis=1, keepdims=True)
        stats_ref[0] = jnp.concatenate([s, sq], axis=1)
    return body


def _bn_body(y_ref, scale_ref, shift_ref, o_ref):
    o_ref[...] = y_ref[...] * scale_ref[...] + shift_ref[...]


def kernel(x_nchw, w_oihw, gamma, beta):
    eps = 1e-5
    N, C, H, W = x_nchw.shape
    Cout, _, KH, KW = w_oihw.shape
    FW = W + 8
    FH = H + 4
    XL = FH * FW
    YL = H * FW
    PL = H * W
    K = KH * KW * C
    taps = tuple((kh, kw) for kh in range(KH) for kw in range(KW))

    x = jnp.pad(x_nchw, ((0, 0), (0, 0), (1, FH - H - 1), (1, FW - W - 1)))
    x = x.reshape(N, C, XL).astype(jnp.bfloat16)
    w = jnp.transpose(w_oihw, (0, 2, 3, 1)).reshape(Cout, K)
    w = w.astype(jnp.bfloat16)

    y, stats = pl.pallas_call(
        _conv_stats_body(H, W, FW, C, K, YL, taps),
        out_shape=(jax.ShapeDtypeStruct((N, Cout, PL), jnp.float32),
                   jax.ShapeDtypeStruct((N, Cout, 2), jnp.float32)),
        grid=(N,),
        in_specs=[pl.BlockSpec((1, C, XL), lambda i: (i, 0, 0)),
                  pl.BlockSpec((Cout, K), lambda i: (0, 0))],
        out_specs=(pl.BlockSpec((1, Cout, PL), lambda i: (i, 0, 0)),
                   pl.BlockSpec((1, Cout, 2), lambda i: (i, 0, 0))),
        scratch_shapes=[pltpu.VMEM((K, YL), jnp.bfloat16)],
        compiler_params=pltpu.CompilerParams(
            dimension_semantics=("parallel",),
            vmem_limit_bytes=64 * 1024 * 1024),
    )(x, w)

    m = N * H * W
    sums = jnp.sum(stats[:, :, 0], axis=0)
    sumsq = jnp.sum(stats[:, :, 1], axis=0)
    mean = sums / m
    var = jnp.maximum(sumsq / m - mean * mean, 0.0)
    scale = gamma.astype(jnp.float32) * lax.rsqrt(var + eps)
    shift = beta.astype(jnp.float32) - mean * scale

    B = next(b for b in (4, 2, 1) if N % b == 0)
    out_flat = pl.pallas_call(
        _bn_body,
        out_shape=jax.ShapeDtypeStruct((N, Cout, PL), jnp.float32),
        grid=(N // B,),
        in_specs=[pl.BlockSpec((B, Cout, PL), lambda i: (i, 0, 0)),
                  pl.BlockSpec((Cout, 1), lambda i: (0, 0)),
                  pl.BlockSpec((Cout, 1), lambda i: (0, 0))],
        out_specs=pl.BlockSpec((B, Cout, PL), lambda i: (i, 0, 0)),
        compiler_params=pltpu.CompilerParams(
            dimension_semantics=("parallel",),
            vmem_limit_bytes=64 * 1024 * 1024),
    )(y, scale.reshape(Cout, 1), shift.reshape(Cout, 1))

    return out_flat.reshape(N, Cout, H, W)


# speedup vs baseline: 1.4425x; 1.0104x over previous
import jax
import jax.numpy as jnp
from jax import lax
from jax.experimental import pallas as pl
from jax.experimental.pallas import tpu as pltpu


def _conv_stats_body(H, W, FW, C, K, YL, taps):
    def body(x_ref, w_ref, y_ref, stats_ref, rhs_ref):
        for t, (kh, kw) in enumerate(taps):
            off = kh * FW + kw
            rhs_ref[pl.ds(t * C, C), :] = x_ref[0, :, pl.ds(off, YL)]
        y = jnp.dot(w_ref[...], rhs_ref[...],
                    preferred_element_type=jnp.float32)
        for r in range(H):
            y_ref[0, :, pl.ds(r * W, W)] = (
                y[:, r * FW:r * FW + W].astype(y_ref.dtype))
        lane = lax.broadcasted_iota(jnp.int32, (1, y.shape[1]), 1)
        ym = y * ((lane % FW) < W).astype(jnp.float32)
        s = jnp.sum(ym, axis=1, keepdims=True)
        sq = jnp.sum(ym * y, axis=1, keepdims=True)
        stats_ref[0] = jnp.concatenate([s, sq], axis=1)
    return body


def _bn_body(y_ref, scale_ref, shift_ref, o_ref):
    o_ref[...] = (y_ref[...].astype(jnp.float32) * scale_ref[...]
                  + shift_ref[...])


def kernel(x_nchw, w_oihw, gamma, beta):
    eps = 1e-5
    N, C, H, W = x_nchw.shape
    Cout, _, KH, KW = w_oihw.shape
    FW = W + 8
    FH = H + 4
    XL = FH * FW
    YL = H * FW
    PL = H * W
    K = KH * KW * C
    taps = tuple((kh, kw) for kh in range(KH) for kw in range(KW))

    x = jnp.pad(x_nchw, ((0, 0), (0, 0), (1, FH - H - 1), (1, FW - W - 1)))
    x = x.reshape(N, C, XL).astype(jnp.bfloat16)
    w = jnp.transpose(w_oihw, (0, 2, 3, 1)).reshape(Cout, K)
    w = w.astype(jnp.bfloat16)

    y, stats = pl.pallas_call(
        _conv_stats_body(H, W, FW, C, K, YL, taps),
        out_shape=(jax.ShapeDtypeStruct((N, Cout, PL), jnp.bfloat16),
                   jax.ShapeDtypeStruct((N, Cout, 2), jnp.float32)),
        grid=(N,),
        in_specs=[pl.BlockSpec((1, C, XL), lambda i: (i, 0, 0)),
                  pl.BlockSpec((Cout, K), lambda i: (0, 0))],
        out_specs=(pl.BlockSpec((1, Cout, PL), lambda i: (i, 0, 0)),
                   pl.BlockSpec((1, Cout, 2), lambda i: (i, 0, 0))),
        scratch_shapes=[pltpu.VMEM((K, YL), jnp.bfloat16)],
        compiler_params=pltpu.CompilerParams(
            dimension_semantics=("parallel",),
            vmem_limit_bytes=64 * 1024 * 1024),
    )(x, w)

    m = N * H * W
    sums = jnp.sum(stats[:, :, 0], axis=0)
    sumsq = jnp.sum(stats[:, :, 1], axis=0)
    mean = sums / m
    var = jnp.maximum(sumsq / m - mean * mean, 0.0)
    scale = gamma.astype(jnp.float32) * lax.rsqrt(var + eps)
    shift = beta.astype(jnp.float32) - mean * scale

    B = next(b for b in (4, 2, 1) if N % b == 0)
    out_flat = pl.pallas_call(
        _bn_body,
        out_shape=jax.ShapeDtypeStruct((N, Cout, PL), jnp.float32),
        grid=(N // B,),
        in_specs=[pl.BlockSpec((B, Cout, PL), lambda i: (i, 0, 0)),
                  pl.BlockSpec((Cout, 1), lambda i: (0, 0)),
                  pl.BlockSpec((Cout, 1), lambda i: (0, 0))],
        out_specs=pl.BlockSpec((B, Cout, PL), lambda i: (i, 0, 0)),
        compiler_params=pltpu.CompilerParams(
            dimension_semantics=("parallel",),
            vmem_limit_bytes=64 * 1024 * 1024),
    )(y, scale.reshape(Cout, 1), shift.reshape(Cout, 1))

    return out_flat.reshape(N, Cout, H, W)


# speedup vs baseline: 1.5854x; 1.0990x over previous
import jax
import jax.numpy as jnp
from jax import lax
from jax.experimental import pallas as pl
from jax.experimental.pallas import tpu as pltpu


def _conv_stats_body(H, W, FW, C, K, YL, taps):
    def body(x_ref, w_ref, y_ref, stats_ref, frame_ref, rhs_ref):
        frame_ref[...] = jnp.zeros_like(frame_ref)
        for r in range(H):
            frame_ref[:, pl.ds((r + 1) * FW + 1, W)] = (
                x_ref[0, :, pl.ds(r * W, W)].astype(frame_ref.dtype))
        for t, (kh, kw) in enumerate(taps):
            off = kh * FW + kw
            rhs_ref[pl.ds(t * C, C), :] = frame_ref[:, pl.ds(off, YL)]
        y = jnp.dot(w_ref[...], rhs_ref[...],
                    preferred_element_type=jnp.float32)
        for r in range(H):
            y_ref[0, :, pl.ds(r * W, W)] = (
                y[:, r * FW:r * FW + W].astype(y_ref.dtype))
        lane = lax.broadcasted_iota(jnp.int32, (1, y.shape[1]), 1)
        ym = y * ((lane % FW) < W).astype(jnp.float32)
        s = jnp.sum(ym, axis=1, keepdims=True)
        sq = jnp.sum(ym * y, axis=1, keepdims=True)
        stats_ref[0] = jnp.concatenate([s, sq], axis=1)
    return body


def _bn_body(y_ref, scale_ref, shift_ref, o_ref):
    o_ref[...] = (y_ref[...].astype(jnp.float32) * scale_ref[...]
                  + shift_ref[...])


def kernel(x_nchw, w_oihw, gamma, beta):
    eps = 1e-5
    N, C, H, W = x_nchw.shape
    Cout, _, KH, KW = w_oihw.shape
    FW = W + 8
    FH = H + 4
    XL = FH * FW
    YL = H * FW
    PL = H * W
    K = KH * KW * C
    taps = tuple((kh, kw) for kh in range(KH) for kw in range(KW))

    x = x_nchw.reshape(N, C, PL)
    w = jnp.transpose(w_oihw, (0, 2, 3, 1)).reshape(Cout, K)
    w = w.astype(jnp.bfloat16)

    y, stats = pl.pallas_call(
        _conv_stats_body(H, W, FW, C, K, YL, taps),
        out_shape=(jax.ShapeDtypeStruct((N, Cout, PL), jnp.bfloat16),
                   jax.ShapeDtypeStruct((N, Cout, 2), jnp.float32)),
        grid=(N,),
        in_specs=[pl.BlockSpec((1, C, PL), lambda i: (i, 0, 0)),
                  pl.BlockSpec((Cout, K), lambda i: (0, 0))],
        out_specs=(pl.BlockSpec((1, Cout, PL), lambda i: (i, 0, 0)),
                   pl.BlockSpec((1, Cout, 2), lambda i: (i, 0, 0))),
        scratch_shapes=[pltpu.VMEM((C, XL), jnp.bfloat16),
                        pltpu.VMEM((K, YL), jnp.bfloat16)],
        compiler_params=pltpu.CompilerParams(
            dimension_semantics=("parallel",),
            vmem_limit_bytes=64 * 1024 * 1024),
    )(x, w)

    m = N * H * W
    sums = jnp.sum(stats[:, :, 0], axis=0)
    sumsq = jnp.sum(stats[:, :, 1], axis=0)
    mean = sums / m
    var = jnp.maximum(sumsq / m - mean * mean, 0.0)
    scale = gamma.astype(jnp.float32) * lax.rsqrt(var + eps)
    shift = beta.astype(jnp.float32) - mean * scale

    B = next(b for b in (4, 2, 1) if N % b == 0)
    out_flat = pl.pallas_call(
        _bn_body,
        out_shape=jax.ShapeDtypeStruct((N, Cout, PL), jnp.float32),
        grid=(N // B,),
        in_specs=[pl.BlockSpec((B, Cout, PL), lambda i: (i, 0, 0)),
                  pl.BlockSpec((Cout, 1), lambda i: (0, 0)),
                  pl.BlockSpec((Cout, 1), lambda i: (0, 0))],
        out_specs=pl.BlockSpec((B, Cout, PL), lambda i: (i, 0, 0)),
        compiler_params=pltpu.CompilerParams(
            dimension_semantics=("parallel",),
            vmem_limit_bytes=64 * 1024 * 1024),
    )(y, scale.reshape(Cout, 1), shift.reshape(Cout, 1))

    return out_flat.reshape(N, Cout, H, W)
